# baseline (speedup 1.0000x reference)
import jax
import jax.numpy as jnp
from jax import lax
from jax.experimental import pallas as pl
from jax.experimental.pallas import tpu as pltpu
from jax.experimental.pallas import tpu_sc as plsc

EMB = 300
NC, NS = 2, 16
NW = NC * NS
SEQ = 50
CHUNK = 200
PLANES = CHUNK // SEQ
NBUF = 2


def _body(table_hbm, tail_hbm, idx_hbm, out_hbm, out2_hbm,
          idx_v, a0, b0_, a1, b1_, g0, g1, w0, w1):
    n_per_w = idx_hbm.shape[0] // NW
    nchunk = n_per_w // CHUNK
    wid = lax.axis_index("s") * NC + lax.axis_index("c")
    pltpu.sync_copy(idx_hbm.at[pl.ds(wid * n_per_w, n_per_w)], idx_v)
    plane0 = wid * (nchunk * PLANES)

    def rows(chunk):
        return idx_v.at[pl.ds(chunk * CHUNK, CHUNK)]

    bufs = ((a0, b0_, g0, w0), (a1, b1_, g1, w1))

    def fire(chunk, s):
        a_v, b_v, gsem, _ = bufs[s]
        pltpu.async_copy(table_hbm.at[rows(chunk), pl.ds(0, 128)], a_v, gsem)
        pltpu.async_copy(table_hbm.at[rows(chunk), pl.ds(128, 128)], b_v, gsem)

    def drain(s):
        a_v, b_v, gsem, _ = bufs[s]
        dummy = table_hbm.at[pl.ds(0, CHUNK), pl.ds(0, 128)]
        pltpu.make_async_copy(dummy, a_v, gsem).wait()
        pltpu.make_async_copy(dummy, b_v, gsem).wait()

    def write_descs(chunk, s):
        a_v, b_v, _, wsem = bufs[s]
        for p in range(PLANES):
            bk = plane0 + chunk * PLANES + p
            sl = pl.ds(SEQ * p, SEQ)
            yield a_v.at[sl], out_hbm.at[bk, :, pl.ds(0, 128)], wsem
            yield b_v.at[sl], out_hbm.at[bk, :, pl.ds(128, 128)], wsem

    for s in range(NBUF):
        fire(s, s)

    @pl.loop(0, nchunk, step=NBUF)
    def _(t):
        for s in range(NBUF):
            chunk = t + s
            drain(s)
            for src, dst, wsem in write_descs(chunk, s):
                pltpu.async_copy(src, dst, wsem)
            for src, dst, wsem in write_descs(chunk, s):
                pltpu.make_async_copy(src, dst, wsem).wait()

            @pl.when(chunk + NBUF < nchunk)
            def _():
                fire(chunk + NBUF, s)

    cb = ((a0, g0, w0), (a1, g1, w1))

    def fire_c(chunk, s):
        c_v, gsem, _ = cb[s]
        pltpu.async_copy(tail_hbm.at[rows(chunk)], c_v, gsem)

    def write_descs_c(chunk, s):
        c_v, _, wsem = cb[s]
        for p in range(PLANES):
            bk = plane0 + chunk * PLANES + p
            yield c_v.at[pl.ds(SEQ * p, SEQ)], out2_hbm.at[bk], wsem

    for s in range(NBUF):
        fire_c(s, s)

    @pl.loop(0, nchunk, step=NBUF)
    def _(t):
        for s in range(NBUF):
            chunk = t + s
            c_v, gsem, _ = cb[s]
            pltpu.make_async_copy(tail_hbm.at[pl.ds(0, CHUNK)],
                                  c_v, gsem).wait()
            for src, dst, wsem in write_descs_c(chunk, s):
                pltpu.async_copy(src, dst, wsem)
            for src, dst, wsem in write_descs_c(chunk, s):
                pltpu.make_async_copy(src, dst, wsem).wait()

            @pl.when(chunk + NBUF < nchunk)
            def _():
                fire_c(chunk + NBUF, s)


def kernel(word_input, word_embedding):
    batch, seq = word_input.shape
    n_per_w = batch * seq // NW
    idx = word_input.astype(jnp.int32).reshape(batch * seq)
    tail = jnp.pad(word_embedding[:, 256:], ((0, 0), (0, 84)))
    mesh = plsc.VectorSubcoreMesh(core_axis_name="c", subcore_axis_name="s")

    k = pl.kernel(
        _body,
        out_type=(
            jax.ShapeDtypeStruct((batch, seq, EMB), jnp.float32),
            jax.ShapeDtypeStruct((batch, seq, 128), jnp.float32),
        ),
        mesh=mesh,
        scratch_types=[
            pltpu.VMEM((n_per_w,), jnp.int32),
            pltpu.VMEM((CHUNK, 128), jnp.float32),
            pltpu.VMEM((CHUNK, 128), jnp.float32),
            pltpu.VMEM((CHUNK, 128), jnp.float32),
            pltpu.VMEM((CHUNK, 128), jnp.float32),
            pltpu.SemaphoreType.DMA,
            pltpu.SemaphoreType.DMA,
            pltpu.SemaphoreType.DMA,
            pltpu.SemaphoreType.DMA,
        ],
    )
    out_main, out2 = k(word_embedding, tail, idx)

    def _merge(m_hbm, t_ref, o_ref):
        del m_hbm
        o_ref[...] = t_ref[...]

    bb = 64
    return pl.pallas_call(
        _merge,
        grid=(batch // bb,),
        in_specs=[
            pl.BlockSpec(memory_space=pl.ANY),
            pl.BlockSpec((bb, seq, 128), lambda i: (i, 0, 0)),
        ],
        out_specs=pl.BlockSpec((bb, seq, 128), lambda i: (i, 0, 2)),
        out_shape=jax.ShapeDtypeStruct((batch, seq, EMB), jnp.float32),
        input_output_aliases={0: 0},
    )(out_main, out2)


# speedup vs baseline: 1.0592x; 1.0592x over previous
import jax
import jax.numpy as jnp
from jax import lax
from jax.experimental import pallas as pl
from jax.experimental.pallas import tpu as pltpu
from jax.experimental.pallas import tpu_sc as plsc

EMB = 300
NC, NS = 2, 16
NW = NC * NS
SEQ = 50
CHUNK = 200
PLANES = CHUNK // SEQ
NBUF = 2


def _body(table_hbm, tail_hbm, idx_hbm, out_hbm, out2_hbm,
          idx_v, a0, b0_, a1, b1_, g0, g1, w0, w1):
    n_per_w = idx_hbm.shape[0] // NW
    nchunk = n_per_w // CHUNK
    wid = lax.axis_index("s") * NC + lax.axis_index("c")
    pltpu.sync_copy(idx_hbm.at[pl.ds(wid * n_per_w, n_per_w)], idx_v)
    plane0 = wid * (nchunk * PLANES)

    def rows(chunk):
        return idx_v.at[pl.ds(chunk * CHUNK, CHUNK)]

    bufs = ((a0, b0_, g0, w0), (a1, b1_, g1, w1))

    def fire(chunk, s):
        a_v, b_v, gsem, _ = bufs[s]
        pltpu.async_copy(table_hbm.at[rows(chunk), pl.ds(0, 128)], a_v, gsem)
        pltpu.async_copy(table_hbm.at[rows(chunk), pl.ds(128, 128)], b_v, gsem)

    def drain(s):
        a_v, b_v, gsem, _ = bufs[s]
        dummy = table_hbm.at[pl.ds(0, CHUNK), pl.ds(0, 128)]
        pltpu.make_async_copy(dummy, a_v, gsem).wait()
        pltpu.make_async_copy(dummy, b_v, gsem).wait()

    def write_descs(chunk, s):
        a_v, b_v, _, wsem = bufs[s]
        for p in range(PLANES):
            bk = plane0 + chunk * PLANES + p
            sl = pl.ds(SEQ * p, SEQ)
            yield a_v.at[sl], out_hbm.at[bk, :, pl.ds(0, 128)], wsem
            yield b_v.at[sl], out_hbm.at[bk, :, pl.ds(128, 128)], wsem

    for s in range(NBUF):
        fire(s, s)

    @pl.loop(0, nchunk, step=NBUF)
    def _(t):
        for s in range(NBUF):
            chunk = t + s
            drain(s)
            for src, dst, wsem in write_descs(chunk, s):
                pltpu.async_copy(src, dst, wsem)
            for src, dst, wsem in write_descs(chunk, s):
                pltpu.make_async_copy(src, dst, wsem).wait()

            @pl.when(chunk + NBUF < nchunk)
            def _():
                fire(chunk + NBUF, s)

    cb = ((a0, g0, w0), (a1, g1, w1))

    def fire_c(chunk, s):
        c_v, gsem, _ = cb[s]
        pltpu.async_copy(tail_hbm.at[rows(chunk)], c_v, gsem)

    def write_descs_c(chunk, s):
        c_v, _, wsem = cb[s]
        for p in range(PLANES):
            bk = plane0 + chunk * PLANES + p
            yield c_v.at[pl.ds(SEQ * p, SEQ)], out2_hbm.at[bk], wsem

    for s in range(NBUF):
        fire_c(s, s)

    @pl.loop(0, nchunk, step=NBUF)
    def _(t):
        for s in range(NBUF):
            chunk = t + s
            c_v, gsem, _ = cb[s]
            pltpu.make_async_copy(tail_hbm.at[pl.ds(0, CHUNK)],
                                  c_v, gsem).wait()
            for src, dst, wsem in write_descs_c(chunk, s):
                pltpu.async_copy(src, dst, wsem)
            for src, dst, wsem in write_descs_c(chunk, s):
                pltpu.make_async_copy(src, dst, wsem).wait()

            @pl.when(chunk + NBUF < nchunk)
            def _():
                fire_c(chunk + NBUF, s)


def kernel(word_input, word_embedding):
    batch, seq = word_input.shape
    n_per_w = batch * seq // NW
    idx = word_input.astype(jnp.int32).reshape(batch * seq)
    tail = jnp.pad(word_embedding[:, 256:], ((0, 0), (0, 84)))
    mesh = plsc.VectorSubcoreMesh(core_axis_name="c", subcore_axis_name="s")

    k = pl.kernel(
        _body,
        out_type=(
            jax.ShapeDtypeStruct((batch, seq, EMB), jnp.float32),
            jax.ShapeDtypeStruct((batch, seq, 128), jnp.float32),
        ),
        mesh=mesh,
        scratch_types=[
            pltpu.VMEM((n_per_w,), jnp.int32),
            pltpu.VMEM((CHUNK, 128), jnp.float32),
            pltpu.VMEM((CHUNK, 128), jnp.float32),
            pltpu.VMEM((CHUNK, 128), jnp.float32),
            pltpu.VMEM((CHUNK, 128), jnp.float32),
            pltpu.SemaphoreType.DMA,
            pltpu.SemaphoreType.DMA,
            pltpu.SemaphoreType.DMA,
            pltpu.SemaphoreType.DMA,
        ],
    )
    out_main, out2 = k(word_embedding, tail, idx)
    return lax.dynamic_update_slice(out_main, out2[:, :, :44], (0, 0, 256))
